# baseline (speedup 1.0000x reference)
import jax
import jax.numpy as jnp
from jax.experimental import pallas as pl
from jax.experimental.pallas import tpu as pltpu


def _pool_body(x_ref, w_ref, o_ref):
    x = x_ref[...]
    bb = x.shape[0]
    s = jnp.sum(x, axis=-1)
    inv = jnp.float32(1.0 / x.shape[-1])
    i = pl.program_id(0)
    o_ref[pl.ds(i * bb, bb), :] = jnp.dot(
        s * inv, w_ref[...], preferred_element_type=jnp.float32)


def _routing_body(l_ref, coef_ref, g_ref, loss_ref):
    logits = l_ref[...]
    B, M = logits.shape
    col = jax.lax.broadcasted_iota(jnp.int32, (B, M), 1)
    big = jnp.int32(M)

    m1 = jnp.max(logits, axis=1, keepdims=True)
    idx1 = jnp.min(jnp.where(logits == m1, col, big), axis=1, keepdims=True)
    masked = jnp.where(col == idx1, -jnp.inf, logits)
    m2 = jnp.max(masked, axis=1, keepdims=True)
    idx2 = jnp.min(jnp.where(masked == m2, col, big), axis=1, keepdims=True)

    e = jnp.exp(m2 - m1)
    denom = 1.0 + e
    g1 = 1.0 / denom
    g2 = e / denom
    gates = (jnp.where(col == idx1, g1, 0.0)
             + jnp.where(col == idx2, g2, 0.0))
    g_ref[...] = gates

    imp = jnp.sum(gates, axis=0, keepdims=True)
    load = jnp.sum((gates > 0.0).astype(jnp.float32), axis=0,
                   keepdims=True)

    def cv_sq(x):
        mean = jnp.sum(x) * jnp.float32(1.0 / M)
        var = jnp.sum((x - mean) ** 2) * jnp.float32(1.0 / (M - 1))
        return var / (mean * mean + jnp.float32(1e-10))

    loss_ref[0, 0] = (cv_sq(imp) + cv_sq(load)) * coef_ref[0]


def kernel(feats, w_gate, w_noise, loss_coef=0.01, noise_epsilon=0.01):
    B, C, H, W = feats.shape
    S = H * W
    M = w_gate.shape[1]
    x = feats.reshape(B, C, S)
    BB = 4

    logits = pl.pallas_call(
        _pool_body,
        grid=(B // BB,),
        in_specs=[
            pl.BlockSpec((BB, C, S), lambda i: (i, 0, 0)),
            pl.BlockSpec((C, M), lambda i: (0, 0)),
        ],
        out_specs=pl.BlockSpec((B, M), lambda i: (0, 0)),
        out_shape=jax.ShapeDtypeStruct((B, M), jnp.float32),
    )(x, w_gate)

    coef = jnp.reshape(jnp.asarray(loss_coef, jnp.float32), (1,))
    gates, loss = pl.pallas_call(
        _routing_body,
        in_specs=[
            pl.BlockSpec(memory_space=pltpu.VMEM),
            pl.BlockSpec(memory_space=pltpu.SMEM),
        ],
        out_specs=[
            pl.BlockSpec(memory_space=pltpu.VMEM),
            pl.BlockSpec(memory_space=pltpu.SMEM),
        ],
        out_shape=[
            jax.ShapeDtypeStruct((B, M), jnp.float32),
            jax.ShapeDtypeStruct((1, 1), jnp.float32),
        ],
    )(logits, coef)

    return gates, loss[0, 0]
